# Initial kernel scaffold; baseline (speedup 1.0000x reference)
import jax
import jax.numpy as jnp
from jax.experimental import pallas as pl


def kernel(x, edge_index, edge_type, W1_rel, W1_root, b1, W2_rel, W2_root, b2):
    raise NotImplementedError("write your pallas kernel here")


# speedup vs baseline: 8.2063x; 8.2063x over previous
import functools

import jax
import jax.numpy as jnp
from jax import lax
from jax.experimental import pallas as pl
from jax.experimental.pallas import tpu as pltpu
from jax.experimental.pallas import tpu_sc as plsc

N = 10000
E = 320000
F_IN = 128
H = 64
C = 64
R = 4

NS = 16
NC = 2
RN = R * N
TROWS = 40960
JUNK = RN
KCH = 128
SLOTS = 327680
NCHUNK = SLOTS // KCH
ROWS_PER_TILE = TROWS // NS
CW = 8
HW = 32

_i32 = jnp.int32
_f32 = jnp.float32


def _tc_pre_body(x_ref, wrel_ref, wroot_ref, b_ref, m_ref, root_ref):
    xb = x_ref[...]
    for r in range(R):
        m_ref[r] = jnp.dot(xb, wrel_ref[r], preferred_element_type=_f32)
    root_ref[...] = (
        jnp.dot(xb, wroot_ref[...], preferred_element_type=_f32) + b_ref[...]
    )


def _tc_pre(x, w_rel, w_root, b):
    f = x.shape[1]
    return pl.pallas_call(
        _tc_pre_body,
        grid=(10,),
        in_specs=[
            pl.BlockSpec((1000, f), lambda i: (i, 0)),
            pl.BlockSpec((R, f, H), lambda i: (0, 0, 0)),
            pl.BlockSpec((f, H), lambda i: (0, 0)),
            pl.BlockSpec((1, H), lambda i: (0, 0)),
        ],
        out_specs=[
            pl.BlockSpec((R, 1000, H), lambda i: (0, i, 0)),
            pl.BlockSpec((1000, H), lambda i: (i, 0)),
        ],
        out_shape=[
            jax.ShapeDtypeStruct((R, N, H), _f32),
            jax.ShapeDtypeStruct((N, H), _f32),
        ],
    )(x, w_rel, w_root, b.reshape(1, H))


def _tc_mid_body(root1_ref, s_ref, cnt_ref, wrel_ref, wroot_ref, b_ref,
                 emb_ref, m_ref, root2_ref):
    cval = cnt_ref[...]
    acc = root1_ref[...]
    for r in range(R):
        d = jnp.maximum(cval[:, r:r + 1] + cval[:, R + r:R + r + 1], 1.0)
        acc = acc + s_ref[r] / d
    emb = jnp.maximum(acc, 0.0)
    emb_ref[...] = emb
    for r in range(R):
        m_ref[r] = jnp.dot(emb, wrel_ref[r], preferred_element_type=_f32)
    root2_ref[...] = (
        jnp.dot(emb, wroot_ref[...], preferred_element_type=_f32) + b_ref[...]
    )


def _tc_mid(root1, s_full, cnt8, w_rel, w_root, b):
    return pl.pallas_call(
        _tc_mid_body,
        grid=(10,),
        in_specs=[
            pl.BlockSpec((1000, H), lambda i: (i, 0)),
            pl.BlockSpec((R, 1000, H), lambda i: (0, i, 0)),
            pl.BlockSpec((1000, 2 * R), lambda i: (i, 0)),
            pl.BlockSpec((R, H, C), lambda i: (0, 0, 0)),
            pl.BlockSpec((H, C), lambda i: (0, 0)),
            pl.BlockSpec((1, C), lambda i: (0, 0)),
        ],
        out_specs=[
            pl.BlockSpec((1000, H), lambda i: (i, 0)),
            pl.BlockSpec((R, 1000, C), lambda i: (0, i, 0)),
            pl.BlockSpec((1000, C), lambda i: (i, 0)),
        ],
        out_shape=[
            jax.ShapeDtypeStruct((N, H), _f32),
            jax.ShapeDtypeStruct((R, N, C), _f32),
            jax.ShapeDtypeStruct((N, C), _f32),
        ],
    )(root1, s_full, cnt8, w_rel, w_root, b.reshape(1, C))


def _tc_post_body(root2_ref, s_ref, cnt_ref, out_ref):
    cval = cnt_ref[...]
    logits = root2_ref[...]
    for r in range(R):
        d = jnp.maximum(cval[:, r:r + 1] + cval[:, R + r:R + r + 1], 1.0)
        logits = logits + s_ref[r] / d
    m = jnp.max(logits, axis=1, keepdims=True)
    sh = logits - m
    out_ref[...] = sh - jnp.log(jnp.sum(jnp.exp(sh), axis=1, keepdims=True))


def _tc_post(root2, s_full, cnt8):
    return pl.pallas_call(
        _tc_post_body,
        grid=(10,),
        in_specs=[
            pl.BlockSpec((1000, C), lambda i: (i, 0)),
            pl.BlockSpec((R, 1000, C), lambda i: (0, i, 0)),
            pl.BlockSpec((1000, 2 * R), lambda i: (i, 0)),
        ],
        out_specs=pl.BlockSpec((1000, C), lambda i: (i, 0)),
        out_shape=jax.ShapeDtypeStruct((N, C), _f32),
    )(root2, s_full, cnt8)


_SC_MESH = plsc.VectorSubcoreMesh(core_axis_name="c", subcore_axis_name="s")
_SC_PARAMS = pltpu.CompilerParams(use_tc_tiling_on_sc=False)


@functools.partial(
    pl.kernel,
    out_type=jax.ShapeDtypeStruct((NC, TROWS, CW), _f32),
    mesh=_SC_MESH,
    compiler_params=_SC_PARAMS,
    scratch_types=[
        pltpu.VMEM((3 * KCH,), _i32),
        pltpu.VMEM((KCH,), _i32),
        pltpu.VMEM((KCH, CW), _f32),
        pltpu.VMEM_SHARED((TROWS, CW), _f32),
    ],
)
def _sc_count(epack_hbm, zc_hbm, ones_hbm, out_hbm, ebuf, dbuf, onesb, ctab):
    cid = lax.axis_index("c")
    sid = lax.axis_index("s")
    w = cid * NS + sid
    base = sid * ROWS_PER_TILE
    pltpu.sync_copy(zc_hbm, ctab.at[pl.ds(base, ROWS_PER_TILE)])
    pltpu.sync_copy(ones_hbm, onesb)
    plsc.subcore_barrier()

    def body(j, carry):
        row = w * (NCHUNK // (NC * NS)) + j
        pltpu.sync_copy(epack_hbm.at[row], ebuf)
        for v in range(KCH // 16):
            d = ebuf[pl.ds(KCH + v * 16, 16)]
            e = ebuf[pl.ds(2 * KCH + v * 16, 16)]
            dbuf[pl.ds(v * 16, 16)] = e * N + d
        pltpu.sync_copy(onesb, ctab.at[dbuf], add=True)
        return carry

    lax.fori_loop(0, NCHUNK // (NC * NS), body, 0)
    plsc.subcore_barrier()
    pltpu.sync_copy(
        ctab.at[pl.ds(base, ROWS_PER_TILE)],
        out_hbm.at[cid, pl.ds(base, ROWS_PER_TILE)],
    )


@functools.partial(
    pl.kernel,
    out_type=jax.ShapeDtypeStruct((NC, TROWS, HW), _f32),
    mesh=_SC_MESH,
    compiler_params=_SC_PARAMS,
    scratch_types=[
        pltpu.VMEM((3 * KCH,), _i32),
        pltpu.VMEM((KCH,), _i32),
        pltpu.VMEM((KCH,), _i32),
        pltpu.VMEM((KCH, HW), _f32),
        pltpu.VMEM_SHARED((TROWS, HW), _f32),
        pltpu.SemaphoreType.DMA,
    ],
)
def _sc_agg(yt_hbm, epack_hbm, zt_hbm, out_hbm,
            ebuf, gbuf, dbuf, rows, stab, sem):
    cid = lax.axis_index("c")
    sid = lax.axis_index("s")
    base = sid * ROWS_PER_TILE
    coff = cid * RN
    pltpu.sync_copy(zt_hbm, stab.at[pl.ds(base, ROWS_PER_TILE)])
    plsc.subcore_barrier()

    def body(j, carry):
        row = sid * (NCHUNK // NS) + j
        pltpu.sync_copy(epack_hbm.at[row], ebuf)
        for v in range(KCH // 16):
            s = ebuf[pl.ds(v * 16, 16)]
            d = ebuf[pl.ds(KCH + v * 16, 16)]
            e = ebuf[pl.ds(2 * KCH + v * 16, 16)]
            en = e * N
            gbuf[pl.ds(v * 16, 16)] = en + s + coff
            dbuf[pl.ds(v * 16, 16)] = en + d
        pltpu.async_copy(yt_hbm.at[gbuf], rows, sem).wait()
        pltpu.sync_copy(rows, stab.at[dbuf], add=True)
        return carry

    lax.fori_loop(0, NCHUNK // NS, body, 0)
    plsc.subcore_barrier()
    pltpu.sync_copy(
        stab.at[pl.ds(base, ROWS_PER_TILE)],
        out_hbm.at[cid, pl.ds(base, ROWS_PER_TILE)],
    )


def _assemble_s(s_part):
    sr = s_part[:, :RN, :].reshape(NC, R, N, HW)
    return jnp.concatenate([sr[0], sr[1]], axis=-1)


def kernel(x, edge_index, edge_type, W1_rel, W1_root, b1, W2_rel, W2_root, b2):
    src = edge_index[0].astype(_i32)
    dst = edge_index[1].astype(_i32)
    et = edge_type.astype(_i32)
    pad = SLOTS - E
    srcp = jnp.concatenate([src, jnp.zeros((pad,), _i32)])
    dstp = jnp.concatenate([dst, jnp.full((pad,), JUNK, _i32)])
    etp = jnp.concatenate([et, jnp.zeros((pad,), _i32)])
    epack = jnp.stack(
        [srcp.reshape(NCHUNK, KCH),
         dstp.reshape(NCHUNK, KCH),
         etp.reshape(NCHUNK, KCH)], axis=1,
    ).reshape(NCHUNK, 3 * KCH)

    zt = jnp.zeros((ROWS_PER_TILE, HW), _f32)
    zc = jnp.zeros((ROWS_PER_TILE, CW), _f32)
    ones = jnp.ones((KCH, CW), _f32)

    cnt_part = _sc_count(epack, zc, ones)
    cnt8 = cnt_part[:, :RN, 0].reshape(NC * R, N).T.reshape(N, NC * R)

    m1, root1 = _tc_pre(x, W1_rel, W1_root, b1)
    yt1 = jnp.moveaxis(m1.reshape(R, N, NC, HW), 2, 0).reshape(NC * RN, HW)
    s1 = _sc_agg(yt1, epack, zt)

    emb, m2, root2 = _tc_mid(root1, _assemble_s(s1), cnt8, W2_rel, W2_root, b2)
    yt2 = jnp.moveaxis(m2.reshape(R, N, NC, HW), 2, 0).reshape(NC * RN, HW)
    s2 = _sc_agg(yt2, epack, zt)

    logsm = _tc_post(root2, _assemble_s(s2), cnt8)
    return (logsm, emb)


# speedup vs baseline: 11.6418x; 1.4186x over previous
import functools

import jax
import jax.numpy as jnp
from jax import lax
from jax.experimental import pallas as pl
from jax.experimental.pallas import tpu as pltpu
from jax.experimental.pallas import tpu_sc as plsc

N = 10000
E = 320000
F_IN = 128
H = 64
C = 64
R = 4

NS = 16
NC = 2
RN = R * N
TROWS = 40960
JUNK = RN
KCH = 128
SLOTS = 327680
NCHUNK = SLOTS // KCH
ROWS_PER_TILE = TROWS // NS
CW = 8
HW = 32

_i32 = jnp.int32
_f32 = jnp.float32


def _tc_pre_body(x_ref, wrel_ref, wroot_ref, b_ref, yt_ref, root_ref):
    xb = x_ref[...]
    for r in range(R):
        m = jnp.dot(xb, wrel_ref[r], preferred_element_type=_f32)
        for c in range(NC):
            yt_ref[c, r] = m[:, c * HW:(c + 1) * HW]
    root_ref[...] = (
        jnp.dot(xb, wroot_ref[...], preferred_element_type=_f32) + b_ref[...]
    )


def _tc_pre(x, w_rel, w_root, b):
    f = x.shape[1]
    return pl.pallas_call(
        _tc_pre_body,
        grid=(10,),
        in_specs=[
            pl.BlockSpec((1000, f), lambda i: (i, 0)),
            pl.BlockSpec((R, f, H), lambda i: (0, 0, 0)),
            pl.BlockSpec((f, H), lambda i: (0, 0)),
            pl.BlockSpec((1, H), lambda i: (0, 0)),
        ],
        out_specs=[
            pl.BlockSpec((NC, R, 1000, HW), lambda i: (0, 0, i, 0)),
            pl.BlockSpec((1000, H), lambda i: (i, 0)),
        ],
        out_shape=[
            jax.ShapeDtypeStruct((NC, R, N, HW), _f32),
            jax.ShapeDtypeStruct((N, H), _f32),
        ],
    )(x, w_rel, w_root, b.reshape(1, H))


def _tc_mid_body(root1_ref, s_ref, cnt_ref, wrel_ref, wroot_ref, b_ref,
                 emb_ref, yt_ref, root2_ref):
    cval = cnt_ref[...]
    acc = root1_ref[...]
    for r in range(R):
        d = jnp.maximum(cval[:, r:r + 1] + cval[:, R + r:R + r + 1], 1.0)
        sc = jnp.concatenate([s_ref[0, r], s_ref[1, r]], axis=-1)
        acc = acc + sc / d
    emb = jnp.maximum(acc, 0.0)
    emb_ref[...] = emb
    for r in range(R):
        m = jnp.dot(emb, wrel_ref[r], preferred_element_type=_f32)
        for c in range(NC):
            yt_ref[c, r] = m[:, c * HW:(c + 1) * HW]
    root2_ref[...] = (
        jnp.dot(emb, wroot_ref[...], preferred_element_type=_f32) + b_ref[...]
    )


def _tc_mid(root1, s_part, cnt8, w_rel, w_root, b):
    return pl.pallas_call(
        _tc_mid_body,
        grid=(10,),
        in_specs=[
            pl.BlockSpec((1000, H), lambda i: (i, 0)),
            pl.BlockSpec((NC, R, 1000, HW), lambda i: (0, 0, i, 0)),
            pl.BlockSpec((1000, 2 * R), lambda i: (i, 0)),
            pl.BlockSpec((R, H, C), lambda i: (0, 0, 0)),
            pl.BlockSpec((H, C), lambda i: (0, 0)),
            pl.BlockSpec((1, C), lambda i: (0, 0)),
        ],
        out_specs=[
            pl.BlockSpec((1000, H), lambda i: (i, 0)),
            pl.BlockSpec((NC, R, 1000, HW), lambda i: (0, 0, i, 0)),
            pl.BlockSpec((1000, C), lambda i: (i, 0)),
        ],
        out_shape=[
            jax.ShapeDtypeStruct((N, H), _f32),
            jax.ShapeDtypeStruct((NC, R, N, HW), _f32),
            jax.ShapeDtypeStruct((N, C), _f32),
        ],
    )(root1, s_part, cnt8, w_rel, w_root, b.reshape(1, C))


def _tc_post_body(root2_ref, s_ref, cnt_ref, out_ref):
    cval = cnt_ref[...]
    logits = root2_ref[...]
    for r in range(R):
        d = jnp.maximum(cval[:, r:r + 1] + cval[:, R + r:R + r + 1], 1.0)
        sc = jnp.concatenate([s_ref[0, r], s_ref[1, r]], axis=-1)
        logits = logits + sc / d
    m = jnp.max(logits, axis=1, keepdims=True)
    sh = logits - m
    out_ref[...] = sh - jnp.log(jnp.sum(jnp.exp(sh), axis=1, keepdims=True))


def _tc_post(root2, s_part, cnt8):
    return pl.pallas_call(
        _tc_post_body,
        grid=(10,),
        in_specs=[
            pl.BlockSpec((1000, C), lambda i: (i, 0)),
            pl.BlockSpec((NC, R, 1000, HW), lambda i: (0, 0, i, 0)),
            pl.BlockSpec((1000, 2 * R), lambda i: (i, 0)),
        ],
        out_specs=pl.BlockSpec((1000, C), lambda i: (i, 0)),
        out_shape=jax.ShapeDtypeStruct((N, C), _f32),
    )(root2, s_part, cnt8)


_SC_MESH = plsc.VectorSubcoreMesh(core_axis_name="c", subcore_axis_name="s")
_SC_PARAMS = pltpu.CompilerParams(use_tc_tiling_on_sc=False)


@functools.partial(
    pl.kernel,
    out_type=jax.ShapeDtypeStruct((NC, TROWS, CW), _f32),
    mesh=_SC_MESH,
    compiler_params=_SC_PARAMS,
    scratch_types=[
        pltpu.VMEM((3 * KCH,), _i32),
        pltpu.VMEM((KCH,), _i32),
        pltpu.VMEM((KCH, CW), _f32),
        pltpu.VMEM_SHARED((TROWS, CW), _f32),
    ],
)
def _sc_count(epack_hbm, zc_hbm, ones_hbm, out_hbm, ebuf, dbuf, onesb, ctab):
    cid = lax.axis_index("c")
    sid = lax.axis_index("s")
    w = cid * NS + sid
    base = sid * ROWS_PER_TILE
    pltpu.sync_copy(zc_hbm, ctab.at[pl.ds(base, ROWS_PER_TILE)])
    pltpu.sync_copy(ones_hbm, onesb)
    plsc.subcore_barrier()

    def body(j, carry):
        row = w * (NCHUNK // (NC * NS)) + j
        pltpu.sync_copy(epack_hbm.at[row], ebuf)
        for v in range(KCH // 16):
            d = ebuf[pl.ds(KCH + v * 16, 16)]
            e = ebuf[pl.ds(2 * KCH + v * 16, 16)]
            dbuf[pl.ds(v * 16, 16)] = e * N + d
        pltpu.sync_copy(onesb, ctab.at[dbuf], add=True)
        return carry

    lax.fori_loop(0, NCHUNK // (NC * NS), body, 0)
    plsc.subcore_barrier()
    pltpu.sync_copy(
        ctab.at[pl.ds(base, ROWS_PER_TILE)],
        out_hbm.at[cid, pl.ds(base, ROWS_PER_TILE)],
    )


_CPT = NCHUNK // NS


@functools.partial(
    pl.kernel,
    out_type=jax.ShapeDtypeStruct((NC, TROWS, HW), _f32),
    mesh=_SC_MESH,
    compiler_params=_SC_PARAMS,
    scratch_types=[
        pltpu.VMEM((2, 3 * KCH), _i32),
        pltpu.VMEM((2, KCH), _i32),
        pltpu.VMEM((2, KCH), _i32),
        pltpu.VMEM((2, KCH, HW), _f32),
        pltpu.VMEM_SHARED((TROWS, HW), _f32),
        pltpu.SemaphoreType.DMA,
        pltpu.SemaphoreType.DMA,
    ],
)
def _sc_agg(yt_hbm, epack_hbm, zt_hbm, out_hbm,
            ebuf, gbuf, dbuf, rows, stab, gsem, esem):
    cid = lax.axis_index("c")
    sid = lax.axis_index("s")
    base = sid * ROWS_PER_TILE
    coff = cid * RN
    row0 = sid * _CPT

    def mk_idx(b):
        for v in range(KCH // 16):
            s = ebuf[b, pl.ds(v * 16, 16)]
            d = ebuf[b, pl.ds(KCH + v * 16, 16)]
            e = ebuf[b, pl.ds(2 * KCH + v * 16, 16)]
            en = e * N
            gbuf[b, pl.ds(v * 16, 16)] = en + s + coff
            dbuf[b, pl.ds(v * 16, 16)] = en + d

    pltpu.sync_copy(zt_hbm, stab.at[pl.ds(base, ROWS_PER_TILE)])
    plsc.subcore_barrier()

    pltpu.sync_copy(epack_hbm.at[row0], ebuf.at[0])
    mk_idx(0)
    pltpu.async_copy(yt_hbm.at[gbuf.at[0]], rows.at[0], gsem)
    pltpu.async_copy(epack_hbm.at[row0 + 1], ebuf.at[1], esem)

    def pair(jj, carry):
        for b in (0, 1):
            j = jj * 2 + b
            nb = 1 - b

            @pl.when(j < _CPT - 1)
            def _prep():
                pltpu.make_async_copy(
                    epack_hbm.at[row0 + j + 1], ebuf.at[nb], esem).wait()
                mk_idx(nb)

            pltpu.make_async_copy(
                yt_hbm.at[gbuf.at[b]], rows.at[b], gsem).wait()

            @pl.when(j < _CPT - 1)
            def _gather():
                pltpu.async_copy(yt_hbm.at[gbuf.at[nb]], rows.at[nb], gsem)

            @pl.when(j < _CPT - 2)
            def _edges():
                pltpu.async_copy(epack_hbm.at[row0 + j + 2], ebuf.at[b], esem)

            pltpu.sync_copy(rows.at[b], stab.at[dbuf.at[b]], add=True)
        return carry

    lax.fori_loop(0, _CPT // 2, pair, 0)
    plsc.subcore_barrier()
    pltpu.sync_copy(
        stab.at[pl.ds(base, ROWS_PER_TILE)],
        out_hbm.at[cid, pl.ds(base, ROWS_PER_TILE)],
    )


def kernel(x, edge_index, edge_type, W1_rel, W1_root, b1, W2_rel, W2_root, b2):
    src = edge_index[0].astype(_i32)
    dst = edge_index[1].astype(_i32)
    et = edge_type.astype(_i32)
    pad = SLOTS - E
    srcp = jnp.concatenate([src, jnp.zeros((pad,), _i32)])
    dstp = jnp.concatenate([dst, jnp.full((pad,), JUNK, _i32)])
    etp = jnp.concatenate([et, jnp.zeros((pad,), _i32)])
    epack = jnp.stack(
        [srcp.reshape(NCHUNK, KCH),
         dstp.reshape(NCHUNK, KCH),
         etp.reshape(NCHUNK, KCH)], axis=1,
    ).reshape(NCHUNK, 3 * KCH)

    zt = jnp.zeros((ROWS_PER_TILE, HW), _f32)
    zc = jnp.zeros((ROWS_PER_TILE, CW), _f32)
    ones = jnp.ones((KCH, CW), _f32)

    cnt_part = _sc_count(epack, zc, ones)
    cnt8 = cnt_part[:, :RN, 0].reshape(NC * R, N).T.reshape(N, NC * R)

    yt1, root1 = _tc_pre(x, W1_rel, W1_root, b1)
    s1 = _sc_agg(yt1.reshape(NC * RN, HW), epack, zt)
    s1v = s1[:, :RN, :].reshape(NC, R, N, HW)

    emb, yt2, root2 = _tc_mid(root1, s1v, cnt8, W2_rel, W2_root, b2)
    s2 = _sc_agg(yt2.reshape(NC * RN, HW), epack, zt)
    s2v = s2[:, :RN, :].reshape(NC, R, N, HW)

    logsm = _tc_post(root2, s2v, cnt8)
    return (logsm, emb)


# speedup vs baseline: 13.2595x; 1.1390x over previous
import functools

import jax
import jax.numpy as jnp
from jax import lax
from jax.experimental import pallas as pl
from jax.experimental.pallas import tpu as pltpu
from jax.experimental.pallas import tpu_sc as plsc

N = 10000
E = 320000
F_IN = 128
H = 64
C = 64
R = 4

NS = 16
NC = 2
RN = R * N
TROWS = 40960
JUNK = RN
KCH = 256
SLOTS = 327680
NCHUNK = SLOTS // KCH
ROWS_PER_TILE = TROWS // NS
CW = 8
HW = 32

_i32 = jnp.int32
_f32 = jnp.float32


def _tc_pre_body(x_ref, wrel_ref, wroot_ref, b_ref, yt_ref, root_ref):
    xb = x_ref[...]
    for r in range(R):
        m = jnp.dot(xb, wrel_ref[r], preferred_element_type=_f32)
        for c in range(NC):
            yt_ref[c, r] = m[:, c * HW:(c + 1) * HW]
    root_ref[...] = (
        jnp.dot(xb, wroot_ref[...], preferred_element_type=_f32) + b_ref[...]
    )


def _tc_pre(x, w_rel, w_root, b):
    f = x.shape[1]
    return pl.pallas_call(
        _tc_pre_body,
        grid=(10,),
        in_specs=[
            pl.BlockSpec((1000, f), lambda i: (i, 0)),
            pl.BlockSpec((R, f, H), lambda i: (0, 0, 0)),
            pl.BlockSpec((f, H), lambda i: (0, 0)),
            pl.BlockSpec((1, H), lambda i: (0, 0)),
        ],
        out_specs=[
            pl.BlockSpec((NC, R, 1000, HW), lambda i: (0, 0, i, 0)),
            pl.BlockSpec((1000, H), lambda i: (i, 0)),
        ],
        out_shape=[
            jax.ShapeDtypeStruct((NC, R, N, HW), _f32),
            jax.ShapeDtypeStruct((N, H), _f32),
        ],
    )(x, w_rel, w_root, b.reshape(1, H))


def _tc_mid_body(root1_ref, s_ref, cnt_ref, wrel_ref, wroot_ref, b_ref,
                 emb_ref, yt_ref, root2_ref):
    cval = cnt_ref[...]
    acc = root1_ref[...]
    for r in range(R):
        d = jnp.maximum(cval[:, r:r + 1] + cval[:, R + r:R + r + 1], 1.0)
        sc = jnp.concatenate([s_ref[0, r], s_ref[1, r]], axis=-1)
        acc = acc + sc / d
    emb = jnp.maximum(acc, 0.0)
    emb_ref[...] = emb
    for r in range(R):
        m = jnp.dot(emb, wrel_ref[r], preferred_element_type=_f32)
        for c in range(NC):
            yt_ref[c, r] = m[:, c * HW:(c + 1) * HW]
    root2_ref[...] = (
        jnp.dot(emb, wroot_ref[...], preferred_element_type=_f32) + b_ref[...]
    )


def _tc_mid(root1, s_part, cnt8, w_rel, w_root, b):
    return pl.pallas_call(
        _tc_mid_body,
        grid=(10,),
        in_specs=[
            pl.BlockSpec((1000, H), lambda i: (i, 0)),
            pl.BlockSpec((NC, R, 1000, HW), lambda i: (0, 0, i, 0)),
            pl.BlockSpec((1000, 2 * R), lambda i: (i, 0)),
            pl.BlockSpec((R, H, C), lambda i: (0, 0, 0)),
            pl.BlockSpec((H, C), lambda i: (0, 0)),
            pl.BlockSpec((1, C), lambda i: (0, 0)),
        ],
        out_specs=[
            pl.BlockSpec((1000, H), lambda i: (i, 0)),
            pl.BlockSpec((NC, R, 1000, HW), lambda i: (0, 0, i, 0)),
            pl.BlockSpec((1000, C), lambda i: (i, 0)),
        ],
        out_shape=[
            jax.ShapeDtypeStruct((N, H), _f32),
            jax.ShapeDtypeStruct((NC, R, N, HW), _f32),
            jax.ShapeDtypeStruct((N, C), _f32),
        ],
    )(root1, s_part, cnt8, w_rel, w_root, b.reshape(1, C))


def _tc_post_body(root2_ref, s_ref, cnt_ref, out_ref):
    cval = cnt_ref[...]
    logits = root2_ref[...]
    for r in range(R):
        d = jnp.maximum(cval[:, r:r + 1] + cval[:, R + r:R + r + 1], 1.0)
        sc = jnp.concatenate([s_ref[0, r], s_ref[1, r]], axis=-1)
        logits = logits + sc / d
    m = jnp.max(logits, axis=1, keepdims=True)
    sh = logits - m
    out_ref[...] = sh - jnp.log(jnp.sum(jnp.exp(sh), axis=1, keepdims=True))


def _tc_post(root2, s_part, cnt8):
    return pl.pallas_call(
        _tc_post_body,
        grid=(10,),
        in_specs=[
            pl.BlockSpec((1000, C), lambda i: (i, 0)),
            pl.BlockSpec((NC, R, 1000, HW), lambda i: (0, 0, i, 0)),
            pl.BlockSpec((1000, 2 * R), lambda i: (i, 0)),
        ],
        out_specs=pl.BlockSpec((1000, C), lambda i: (i, 0)),
        out_shape=jax.ShapeDtypeStruct((N, C), _f32),
    )(root2, s_part, cnt8)


_SC_MESH = plsc.VectorSubcoreMesh(core_axis_name="c", subcore_axis_name="s")
_SC_PARAMS = pltpu.CompilerParams(use_tc_tiling_on_sc=False)


@functools.partial(
    pl.kernel,
    out_type=jax.ShapeDtypeStruct((NC, TROWS, CW), _f32),
    mesh=_SC_MESH,
    compiler_params=_SC_PARAMS,
    scratch_types=[
        pltpu.VMEM((3 * KCH,), _i32),
        pltpu.VMEM((KCH,), _i32),
        pltpu.VMEM((KCH, CW), _f32),
        pltpu.VMEM_SHARED((TROWS, CW), _f32),
    ],
)
def _sc_count(epack_hbm, zc_hbm, ones_hbm, out_hbm, ebuf, dbuf, onesb, ctab):
    cid = lax.axis_index("c")
    sid = lax.axis_index("s")
    w = cid * NS + sid
    base = sid * ROWS_PER_TILE
    pltpu.sync_copy(zc_hbm, ctab.at[pl.ds(base, ROWS_PER_TILE)])
    pltpu.sync_copy(ones_hbm, onesb)
    plsc.subcore_barrier()

    def body(j, carry):
        row = w * (NCHUNK // (NC * NS)) + j
        pltpu.sync_copy(epack_hbm.at[row], ebuf)
        for v in range(KCH // 16):
            d = ebuf[pl.ds(KCH + v * 16, 16)]
            e = ebuf[pl.ds(2 * KCH + v * 16, 16)]
            dbuf[pl.ds(v * 16, 16)] = e * N + d
        pltpu.sync_copy(onesb, ctab.at[dbuf], add=True)
        return carry

    lax.fori_loop(0, NCHUNK // (NC * NS), body, 0)
    plsc.subcore_barrier()
    pltpu.sync_copy(
        ctab.at[pl.ds(base, ROWS_PER_TILE)],
        out_hbm.at[cid, pl.ds(base, ROWS_PER_TILE)],
    )


_CPT = NCHUNK // NS


@functools.partial(
    pl.kernel,
    out_type=jax.ShapeDtypeStruct((NC, TROWS, HW), _f32),
    mesh=_SC_MESH,
    compiler_params=_SC_PARAMS,
    scratch_types=[
        pltpu.VMEM((2, 3 * KCH), _i32),
        pltpu.VMEM((2, KCH), _i32),
        pltpu.VMEM((2, KCH), _i32),
        pltpu.VMEM((2, KCH, HW), _f32),
        pltpu.VMEM_SHARED((TROWS, HW), _f32),
        pltpu.SemaphoreType.DMA,
        pltpu.SemaphoreType.DMA,
    ],
)
def _sc_agg(yt_hbm, epack_hbm, zt_hbm, out_hbm,
            ebuf, gbuf, dbuf, rows, stab, gsem, esem):
    cid = lax.axis_index("c")
    sid = lax.axis_index("s")
    base = sid * ROWS_PER_TILE
    coff = cid * RN
    row0 = sid * _CPT

    def mk_idx(b):
        for v in range(KCH // 16):
            s = ebuf[b, pl.ds(v * 16, 16)]
            d = ebuf[b, pl.ds(KCH + v * 16, 16)]
            e = ebuf[b, pl.ds(2 * KCH + v * 16, 16)]
            en = e * N
            gbuf[b, pl.ds(v * 16, 16)] = en + s + coff
            dbuf[b, pl.ds(v * 16, 16)] = en + d

    pltpu.sync_copy(zt_hbm, stab.at[pl.ds(base, ROWS_PER_TILE)])
    plsc.subcore_barrier()

    pltpu.sync_copy(epack_hbm.at[row0], ebuf.at[0])
    mk_idx(0)
    pltpu.async_copy(yt_hbm.at[gbuf.at[0]], rows.at[0], gsem)
    pltpu.async_copy(epack_hbm.at[row0 + 1], ebuf.at[1], esem)

    def pair(jj, carry):
        for b in (0, 1):
            j = jj * 2 + b
            nb = 1 - b

            @pl.when(j < _CPT - 1)
            def _prep():
                pltpu.make_async_copy(
                    epack_hbm.at[row0 + j + 1], ebuf.at[nb], esem).wait()
                mk_idx(nb)

            pltpu.make_async_copy(
                yt_hbm.at[gbuf.at[b]], rows.at[b], gsem).wait()

            @pl.when(j < _CPT - 1)
            def _gather():
                pltpu.async_copy(yt_hbm.at[gbuf.at[nb]], rows.at[nb], gsem)

            @pl.when(j < _CPT - 2)
            def _edges():
                pltpu.async_copy(epack_hbm.at[row0 + j + 2], ebuf.at[b], esem)

            pltpu.sync_copy(rows.at[b], stab.at[dbuf.at[b]], add=True)
        return carry

    lax.fori_loop(0, _CPT // 2, pair, 0)
    plsc.subcore_barrier()
    pltpu.sync_copy(
        stab.at[pl.ds(base, ROWS_PER_TILE)],
        out_hbm.at[cid, pl.ds(base, ROWS_PER_TILE)],
    )


def kernel(x, edge_index, edge_type, W1_rel, W1_root, b1, W2_rel, W2_root, b2):
    src = edge_index[0].astype(_i32)
    dst = edge_index[1].astype(_i32)
    et = edge_type.astype(_i32)
    pad = SLOTS - E
    srcp = jnp.concatenate([src, jnp.zeros((pad,), _i32)])
    dstp = jnp.concatenate([dst, jnp.full((pad,), JUNK, _i32)])
    etp = jnp.concatenate([et, jnp.zeros((pad,), _i32)])
    epack = jnp.stack(
        [srcp.reshape(NCHUNK, KCH),
         dstp.reshape(NCHUNK, KCH),
         etp.reshape(NCHUNK, KCH)], axis=1,
    ).reshape(NCHUNK, 3 * KCH)

    zt = jnp.zeros((ROWS_PER_TILE, HW), _f32)
    zc = jnp.zeros((ROWS_PER_TILE, CW), _f32)
    ones = jnp.ones((KCH, CW), _f32)

    cnt_part = _sc_count(epack, zc, ones)
    cnt8 = cnt_part[:, :RN, 0].reshape(NC * R, N).T.reshape(N, NC * R)

    yt1, root1 = _tc_pre(x, W1_rel, W1_root, b1)
    s1 = _sc_agg(yt1.reshape(NC * RN, HW), epack, zt)
    s1v = s1[:, :RN, :].reshape(NC, R, N, HW)

    emb, yt2, root2 = _tc_mid(root1, s1v, cnt8, W2_rel, W2_root, b2)
    s2 = _sc_agg(yt2.reshape(NC * RN, HW), epack, zt)
    s2v = s2[:, :RN, :].reshape(NC, R, N, HW)

    logsm = _tc_post(root2, s2v, cnt8)
    return (logsm, emb)


# speedup vs baseline: 13.8791x; 1.0467x over previous
import functools

import jax
import jax.numpy as jnp
from jax import lax
from jax.experimental import pallas as pl
from jax.experimental.pallas import tpu as pltpu
from jax.experimental.pallas import tpu_sc as plsc

N = 10000
E = 320000
F_IN = 128
H = 64
C = 64
R = 4

NS = 16
NC = 2
RN = R * N
TROWS = 40960
JUNK = RN
KCH = 512
SLOTS = 327680
NCHUNK = SLOTS // KCH
ROWS_PER_TILE = TROWS // NS
CW = 8
HW = 32

_i32 = jnp.int32
_f32 = jnp.float32


def _tc_pre_body(x_ref, wrel_ref, wroot_ref, b_ref, yt_ref, root_ref):
    xb = x_ref[...]
    for r in range(R):
        m = jnp.dot(xb, wrel_ref[r], preferred_element_type=_f32)
        for c in range(NC):
            yt_ref[c, r] = m[:, c * HW:(c + 1) * HW]
    root_ref[...] = (
        jnp.dot(xb, wroot_ref[...], preferred_element_type=_f32) + b_ref[...]
    )


def _tc_pre(x, w_rel, w_root, b):
    f = x.shape[1]
    return pl.pallas_call(
        _tc_pre_body,
        grid=(10,),
        in_specs=[
            pl.BlockSpec((1000, f), lambda i: (i, 0)),
            pl.BlockSpec((R, f, H), lambda i: (0, 0, 0)),
            pl.BlockSpec((f, H), lambda i: (0, 0)),
            pl.BlockSpec((1, H), lambda i: (0, 0)),
        ],
        out_specs=[
            pl.BlockSpec((NC, R, 1000, HW), lambda i: (0, 0, i, 0)),
            pl.BlockSpec((1000, H), lambda i: (i, 0)),
        ],
        out_shape=[
            jax.ShapeDtypeStruct((NC, R, N, HW), _f32),
            jax.ShapeDtypeStruct((N, H), _f32),
        ],
    )(x, w_rel, w_root, b.reshape(1, H))


def _tc_mid_body(root1_ref, s_ref, cnt_ref, wrel_ref, wroot_ref, b_ref,
                 emb_ref, yt_ref, root2_ref):
    cval = cnt_ref[...]
    acc = root1_ref[...]
    for r in range(R):
        d = jnp.maximum(cval[:, r:r + 1] + cval[:, R + r:R + r + 1], 1.0)
        sc = jnp.concatenate([s_ref[0, r], s_ref[1, r]], axis=-1)
        acc = acc + sc / d
    emb = jnp.maximum(acc, 0.0)
    emb_ref[...] = emb
    for r in range(R):
        m = jnp.dot(emb, wrel_ref[r], preferred_element_type=_f32)
        for c in range(NC):
            yt_ref[c, r] = m[:, c * HW:(c + 1) * HW]
    root2_ref[...] = (
        jnp.dot(emb, wroot_ref[...], preferred_element_type=_f32) + b_ref[...]
    )


def _tc_mid(root1, s_part, cnt8, w_rel, w_root, b):
    return pl.pallas_call(
        _tc_mid_body,
        grid=(10,),
        in_specs=[
            pl.BlockSpec((1000, H), lambda i: (i, 0)),
            pl.BlockSpec((NC, R, 1000, HW), lambda i: (0, 0, i, 0)),
            pl.BlockSpec((1000, 2 * R), lambda i: (i, 0)),
            pl.BlockSpec((R, H, C), lambda i: (0, 0, 0)),
            pl.BlockSpec((H, C), lambda i: (0, 0)),
            pl.BlockSpec((1, C), lambda i: (0, 0)),
        ],
        out_specs=[
            pl.BlockSpec((1000, H), lambda i: (i, 0)),
            pl.BlockSpec((NC, R, 1000, HW), lambda i: (0, 0, i, 0)),
            pl.BlockSpec((1000, C), lambda i: (i, 0)),
        ],
        out_shape=[
            jax.ShapeDtypeStruct((N, H), _f32),
            jax.ShapeDtypeStruct((NC, R, N, HW), _f32),
            jax.ShapeDtypeStruct((N, C), _f32),
        ],
    )(root1, s_part, cnt8, w_rel, w_root, b.reshape(1, C))


def _tc_post_body(root2_ref, s_ref, cnt_ref, out_ref):
    cval = cnt_ref[...]
    logits = root2_ref[...]
    for r in range(R):
        d = jnp.maximum(cval[:, r:r + 1] + cval[:, R + r:R + r + 1], 1.0)
        sc = jnp.concatenate([s_ref[0, r], s_ref[1, r]], axis=-1)
        logits = logits + sc / d
    m = jnp.max(logits, axis=1, keepdims=True)
    sh = logits - m
    out_ref[...] = sh - jnp.log(jnp.sum(jnp.exp(sh), axis=1, keepdims=True))


def _tc_post(root2, s_part, cnt8):
    return pl.pallas_call(
        _tc_post_body,
        grid=(10,),
        in_specs=[
            pl.BlockSpec((1000, C), lambda i: (i, 0)),
            pl.BlockSpec((NC, R, 1000, HW), lambda i: (0, 0, i, 0)),
            pl.BlockSpec((1000, 2 * R), lambda i: (i, 0)),
        ],
        out_specs=pl.BlockSpec((1000, C), lambda i: (i, 0)),
        out_shape=jax.ShapeDtypeStruct((N, C), _f32),
    )(root2, s_part, cnt8)


_SC_MESH = plsc.VectorSubcoreMesh(core_axis_name="c", subcore_axis_name="s")
_SC_PARAMS = pltpu.CompilerParams(use_tc_tiling_on_sc=False)


@functools.partial(
    pl.kernel,
    out_type=jax.ShapeDtypeStruct((NC, TROWS, CW), _f32),
    mesh=_SC_MESH,
    compiler_params=_SC_PARAMS,
    scratch_types=[
        pltpu.VMEM((3 * KCH,), _i32),
        pltpu.VMEM((KCH,), _i32),
        pltpu.VMEM((KCH, CW), _f32),
        pltpu.VMEM_SHARED((TROWS, CW), _f32),
    ],
)
def _sc_count(epack_hbm, zc_hbm, ones_hbm, out_hbm, ebuf, dbuf, onesb, ctab):
    cid = lax.axis_index("c")
    sid = lax.axis_index("s")
    w = cid * NS + sid
    base = sid * ROWS_PER_TILE
    pltpu.sync_copy(zc_hbm, ctab.at[pl.ds(base, ROWS_PER_TILE)])
    pltpu.sync_copy(ones_hbm, onesb)
    plsc.subcore_barrier()

    def body(j, carry):
        row = w * (NCHUNK // (NC * NS)) + j
        pltpu.sync_copy(epack_hbm.at[row], ebuf)
        for v in range(KCH // 16):
            d = ebuf[pl.ds(KCH + v * 16, 16)]
            e = ebuf[pl.ds(2 * KCH + v * 16, 16)]
            dbuf[pl.ds(v * 16, 16)] = e * N + d
        pltpu.sync_copy(onesb, ctab.at[dbuf], add=True)
        return carry

    lax.fori_loop(0, NCHUNK // (NC * NS), body, 0)
    plsc.subcore_barrier()
    pltpu.sync_copy(
        ctab.at[pl.ds(base, ROWS_PER_TILE)],
        out_hbm.at[cid, pl.ds(base, ROWS_PER_TILE)],
    )


_CPT = NCHUNK // NS


@functools.partial(
    pl.kernel,
    out_type=jax.ShapeDtypeStruct((NC, TROWS, HW), _f32),
    mesh=_SC_MESH,
    compiler_params=_SC_PARAMS,
    scratch_types=[
        pltpu.VMEM((2, 3 * KCH), _i32),
        pltpu.VMEM((2, KCH), _i32),
        pltpu.VMEM((2, KCH), _i32),
        pltpu.VMEM((2, KCH, HW), _f32),
        pltpu.VMEM_SHARED((TROWS, HW), _f32),
        pltpu.SemaphoreType.DMA,
        pltpu.SemaphoreType.DMA,
    ],
)
def _sc_agg(yt_hbm, epack_hbm, zt_hbm, out_hbm,
            ebuf, gbuf, dbuf, rows, stab, gsem, esem):
    cid = lax.axis_index("c")
    sid = lax.axis_index("s")
    base = sid * ROWS_PER_TILE
    coff = cid * RN
    row0 = sid * _CPT

    def mk_idx(b):
        for v in range(KCH // 16):
            s = ebuf[b, pl.ds(v * 16, 16)]
            d = ebuf[b, pl.ds(KCH + v * 16, 16)]
            e = ebuf[b, pl.ds(2 * KCH + v * 16, 16)]
            en = e * N
            gbuf[b, pl.ds(v * 16, 16)] = en + s + coff
            dbuf[b, pl.ds(v * 16, 16)] = en + d

    pltpu.sync_copy(zt_hbm, stab.at[pl.ds(base, ROWS_PER_TILE)])
    plsc.subcore_barrier()

    pltpu.sync_copy(epack_hbm.at[row0], ebuf.at[0])
    mk_idx(0)
    pltpu.async_copy(yt_hbm.at[gbuf.at[0]], rows.at[0], gsem)
    pltpu.async_copy(epack_hbm.at[row0 + 1], ebuf.at[1], esem)

    def pair(jj, carry):
        for b in (0, 1):
            j = jj * 2 + b
            nb = 1 - b

            @pl.when(j < _CPT - 1)
            def _prep():
                pltpu.make_async_copy(
                    epack_hbm.at[row0 + j + 1], ebuf.at[nb], esem).wait()
                mk_idx(nb)

            pltpu.make_async_copy(
                yt_hbm.at[gbuf.at[b]], rows.at[b], gsem).wait()

            @pl.when(j < _CPT - 1)
            def _gather():
                pltpu.async_copy(yt_hbm.at[gbuf.at[nb]], rows.at[nb], gsem)

            @pl.when(j < _CPT - 2)
            def _edges():
                pltpu.async_copy(epack_hbm.at[row0 + j + 2], ebuf.at[b], esem)

            pltpu.sync_copy(rows.at[b], stab.at[dbuf.at[b]], add=True)
        return carry

    lax.fori_loop(0, _CPT // 2, pair, 0)
    plsc.subcore_barrier()
    pltpu.sync_copy(
        stab.at[pl.ds(base, ROWS_PER_TILE)],
        out_hbm.at[cid, pl.ds(base, ROWS_PER_TILE)],
    )


def kernel(x, edge_index, edge_type, W1_rel, W1_root, b1, W2_rel, W2_root, b2):
    src = edge_index[0].astype(_i32)
    dst = edge_index[1].astype(_i32)
    et = edge_type.astype(_i32)
    pad = SLOTS - E
    srcp = jnp.concatenate([src, jnp.zeros((pad,), _i32)])
    dstp = jnp.concatenate([dst, jnp.full((pad,), JUNK, _i32)])
    etp = jnp.concatenate([et, jnp.zeros((pad,), _i32)])
    epack = jnp.stack(
        [srcp.reshape(NCHUNK, KCH),
         dstp.reshape(NCHUNK, KCH),
         etp.reshape(NCHUNK, KCH)], axis=1,
    ).reshape(NCHUNK, 3 * KCH)

    zt = jnp.zeros((ROWS_PER_TILE, HW), _f32)
    zc = jnp.zeros((ROWS_PER_TILE, CW), _f32)
    ones = jnp.ones((KCH, CW), _f32)

    cnt_part = _sc_count(epack, zc, ones)
    cnt8 = cnt_part[:, :RN, 0].reshape(NC * R, N).T.reshape(N, NC * R)

    yt1, root1 = _tc_pre(x, W1_rel, W1_root, b1)
    s1 = _sc_agg(yt1.reshape(NC * RN, HW), epack, zt)
    s1v = s1[:, :RN, :].reshape(NC, R, N, HW)

    emb, yt2, root2 = _tc_mid(root1, s1v, cnt8, W2_rel, W2_root, b2)
    s2 = _sc_agg(yt2.reshape(NC * RN, HW), epack, zt)
    s2v = s2[:, :RN, :].reshape(NC, R, N, HW)

    logsm = _tc_post(root2, s2v, cnt8)
    return (logsm, emb)
